# baseline (speedup 1.0000x reference)
import functools

import jax
import jax.numpy as jnp
from jax import lax
from jax.experimental import pallas as pl
from jax.experimental.pallas import tpu as pltpu
from jax.experimental.pallas import tpu_sc as plsc

B, S, D, V = 2, 2048, 1024, 30000
T = B * S
THRESHOLD = 128

_NC, _NS = 2, 16
_NW = _NC * _NS
_RW = T // _NW
_CH = 64


def _sc_gather_body(table_hbm, idx_hbm, out_hbm, idx_v, rows_v, sem):
  wid = lax.axis_index("s") * _NC + lax.axis_index("c")
  base = wid * _RW
  for c in range(_RW // _CH):
    off = base + c * _CH
    pltpu.sync_copy(idx_hbm.at[pl.ds(off, _CH)], idx_v)
    pltpu.async_copy(table_hbm.at[idx_v], rows_v, sem).wait()
    pltpu.sync_copy(rows_v, out_hbm.at[pl.ds(off, _CH)])


def _sc_gather(table, idx):
  mesh = plsc.VectorSubcoreMesh(core_axis_name="c", subcore_axis_name="s")
  fn = pl.kernel(
      _sc_gather_body,
      out_type=jax.ShapeDtypeStruct((T, D), jnp.float32),
      mesh=mesh,
      scratch_types=[
          pltpu.VMEM((_CH,), jnp.int32),
          pltpu.VMEM((_CH, D), jnp.float32),
          pltpu.SemaphoreType.DMA,
      ],
  )
  return fn(table, idx)


def _bf(x):
  return x.astype(jnp.bfloat16)


_BM_MM = 512


def _mm_body(x_ref, w_ref, o_ref):
  o_ref[...] = _bf(jnp.dot(_bf(x_ref[...]), w_ref[...],
                           preferred_element_type=jnp.float32))


def _matmul(x, w):
  return pl.pallas_call(
      _mm_body,
      grid=(T // _BM_MM,),
      in_specs=[
          pl.BlockSpec((_BM_MM, D), lambda i: (i, 0)),
          pl.BlockSpec((D, D), lambda i: (0, 0)),
      ],
      out_specs=pl.BlockSpec((_BM_MM, D), lambda i: (i, 0)),
      out_shape=jax.ShapeDtypeStruct((T, D), jnp.bfloat16),
  )(x, _bf(w))


_BM_ADJ = 256


def _adj_w_body(a_ref, s_ref, w_ref, o_ref):
  a = a_ref[0].astype(jnp.float32)
  deg = jnp.clip(jnp.sum(a, axis=1, keepdims=True), 1e-9, None)
  h = jnp.maximum(
      jnp.dot(_bf(a / deg), s_ref[0], preferred_element_type=jnp.float32),
      0.0)
  o_ref[0] = _bf(jnp.dot(_bf(h), w_ref[...],
                         preferred_element_type=jnp.float32))


def _adj_mm_w(adj_bf, sup, w2):
  return pl.pallas_call(
      _adj_w_body,
      grid=(B, S // _BM_ADJ),
      in_specs=[
          pl.BlockSpec((1, _BM_ADJ, S), lambda b, i: (b, i, 0)),
          pl.BlockSpec((1, S, D), lambda b, i: (b, 0, 0)),
          pl.BlockSpec((D, D), lambda b, i: (0, 0)),
      ],
      out_specs=pl.BlockSpec((1, _BM_ADJ, D), lambda b, i: (b, i, 0)),
      out_shape=jax.ShapeDtypeStruct((B, S, D), jnp.bfloat16),
  )(adj_bf, sup, _bf(w2))


def _adj_ln_body(a_ref, s_ref, hs_ref, g_ref, bb_ref, o_ref):
  a = a_ref[0].astype(jnp.float32)
  deg = jnp.clip(jnp.sum(a, axis=1, keepdims=True), 1e-9, None)
  h = jnp.maximum(
      jnp.dot(_bf(a / deg), s_ref[0], preferred_element_type=jnp.float32),
      0.0)
  x = h + hs_ref[0]
  m = jnp.mean(x, axis=1, keepdims=True)
  v = jnp.mean((x - m) ** 2, axis=1, keepdims=True)
  o_ref[0] = _bf((x - m) * lax.rsqrt(v + 1e-5) * g_ref[...] + bb_ref[...])


def _adj_mm_ln(adj_bf, sup, hs, ln_g, ln_b):
  return pl.pallas_call(
      _adj_ln_body,
      grid=(B, S // _BM_ADJ),
      in_specs=[
          pl.BlockSpec((1, _BM_ADJ, S), lambda b, i: (b, i, 0)),
          pl.BlockSpec((1, S, D), lambda b, i: (b, 0, 0)),
          pl.BlockSpec((1, _BM_ADJ, D), lambda b, i: (b, i, 0)),
          pl.BlockSpec((1, D), lambda b, i: (0, 0)),
          pl.BlockSpec((1, D), lambda b, i: (0, 0)),
      ],
      out_specs=pl.BlockSpec((1, _BM_ADJ, D), lambda b, i: (b, i, 0)),
      out_shape=jax.ShapeDtypeStruct((B, S, D), jnp.bfloat16),
  )(adj_bf, sup, hs.reshape(B, S, D), ln_g.reshape(1, D), ln_b.reshape(1, D))


_BM_FUSE = 256


def _gelu(x):
  return x * 0.5 * (1.0 + lax.erf(x * (2.0 ** -0.5)))


def _fuse_body(seq_ref, hs_ref, sh_ref, rw_ref, rb_ref,
               synw_ref, synb_ref, lw_ref, lb_ref,
               semw_ref, semb_ref, cw_ref, cb_ref, o_ref):
  i = pl.program_id(0)
  b = i // (S // _BM_FUSE)
  short = seq_ref[b] <= THRESHOLD

  hs = hs_ref[...]
  hs_bf = _bf(hs)
  shared = sh_ref[...]

  rl = jnp.dot(hs, rw_ref[...], preferred_element_type=jnp.float32) \
      + rb_ref[...]
  col = lax.broadcasted_iota(jnp.int32, rl.shape, 1)
  neg = jnp.float32(-1e9)
  rl = jnp.where(jnp.logical_and(col == 4, short), neg, rl)
  rl = jnp.where(jnp.logical_and(col == 3, jnp.logical_not(short)), neg, rl)
  rl = rl - jnp.max(rl, axis=1, keepdims=True)
  e = jnp.exp(rl)
  probs = e / jnp.sum(e, axis=1, keepdims=True)

  def group_max(lo, n):
    mx = probs[:, lo:lo + 1]
    idx = jnp.zeros_like(mx, dtype=jnp.int32)
    for j in range(1, n):
      p = probs[:, lo + j:lo + j + 1]
      idx = jnp.where(p > mx, j, idx)
      mx = jnp.maximum(mx, p)
    return mx, idx

  syn_p, syn_i = group_max(0, 3)
  sem_p, sem_i = group_max(5, 3)
  len_p = jnp.where(short, probs[:, 3:4], probs[:, 4:5])
  tot = syn_p + len_p + sem_p
  w_syn = syn_p / tot
  w_len = len_p / tot
  w_sem = sem_p / tot

  fused = jnp.zeros((_BM_FUSE, D), jnp.float32)
  for j in range(3):
    eo = _gelu(jnp.dot(shared, synw_ref[j], preferred_element_type=jnp.float32)
               + synb_ref[j:j + 1])
    fused = fused + jnp.where(syn_i == j, w_syn, 0.0) * eo
  lo_ = _gelu(jnp.dot(hs_bf, lw_ref[0], preferred_element_type=jnp.float32)
              + lb_ref[0])
  fused = fused + w_len * lo_
  for j in range(3):
    eo = _gelu(jnp.dot(hs_bf, semw_ref[j], preferred_element_type=jnp.float32)
               + semb_ref[j:j + 1])
    fused = fused + jnp.where(sem_i == j, w_sem, 0.0) * eo

  o_ref[...] = jnp.dot(fused, cw_ref[...],
                       preferred_element_type=jnp.float32) + cb_ref[...]


def _fuse(seq_lengths, hs, shared, router_W, router_b, syn_W, syn_b,
          lens_W, lens_b, sem_W, sem_b, cls_W, cls_b):
  nb = S // _BM_FUSE
  full = lambda shape: pl.BlockSpec(shape, lambda i, s: tuple(0 for _ in shape))
  grid_spec = pltpu.PrefetchScalarGridSpec(
      num_scalar_prefetch=1,
      grid=(T // _BM_FUSE,),
      in_specs=[
          pl.BlockSpec((_BM_FUSE, D), lambda i, s: (i, 0)),
          pl.BlockSpec((_BM_FUSE, D), lambda i, s: (i, 0)),
          full((D, 8)), full((1, 8)),
          full((3, D, D)), full((3, D)),
          pl.BlockSpec(
              (1, D, D),
              lambda i, s: (jnp.where(s[i // nb] <= THRESHOLD, 0, 1), 0, 0)),
          pl.BlockSpec(
              (1, 1, D),
              lambda i, s: (jnp.where(s[i // nb] <= THRESHOLD, 0, 1), 0, 0)),
          full((3, D, D)), full((3, D)),
          full((D, 2)), full((1, 2)),
      ],
      out_specs=pl.BlockSpec((_BM_FUSE, 2), lambda i, s: (i, 0)),
  )
  return pl.pallas_call(
      _fuse_body,
      grid_spec=grid_spec,
      out_shape=jax.ShapeDtypeStruct((T, 2), jnp.float32),
  )(seq_lengths, hs, shared, router_W, router_b.reshape(1, 8),
    _bf(syn_W), syn_b, _bf(lens_W), lens_b.reshape(2, 1, D),
    _bf(sem_W), sem_b, cls_W, cls_b.reshape(1, 2))


def kernel(input_ids, attention_mask, seq_lengths, adj_matrix, emb, router_W,
           router_b, gcn1_W, gcn2_W, ln_g, ln_b, syn_W, syn_b, lenS_W, lenS_b,
           lenL_W, lenL_b, sem_W, sem_b, cls_W, cls_b):
  del attention_mask
  ids = input_ids.reshape(T).astype(jnp.int32)
  hs = _sc_gather(emb, ids)
  adj_bf = adj_matrix.astype(jnp.bfloat16)
  sup1 = _matmul(hs, gcn1_W)
  sup2 = _adj_mm_w(adj_bf, sup1.reshape(B, S, D), gcn2_W)
  shared = _adj_mm_ln(adj_bf, sup2, hs, ln_g, ln_b).reshape(T, D)
  lens_W = jnp.stack([lenS_W, lenL_W])
  lens_b = jnp.stack([lenS_b, lenL_b])
  logits = _fuse(seq_lengths.astype(jnp.int32), hs, shared,
                 router_W, router_b, syn_W, syn_b, lens_W, lens_b,
                 sem_W, sem_b, cls_W, cls_b)
  return logits.reshape(B, S, 2)


# speedup vs baseline: 1.1466x; 1.1466x over previous
import functools

import jax
import jax.numpy as jnp
from jax import lax
from jax.experimental import pallas as pl
from jax.experimental.pallas import tpu as pltpu
from jax.experimental.pallas import tpu_sc as plsc

B, S, D, V = 2, 2048, 1024, 30000
T = B * S
THRESHOLD = 128

_NC, _NS = 2, 16
_NW = _NC * _NS
_ROWS_PER_W = T // _NW
_CH = 64


def _sc_gather_body(table_hbm, idx_hbm, out_hbm, idx_v, rows_v, sem):
  wid = lax.axis_index("s") * _NC + lax.axis_index("c")
  base = wid * _ROWS_PER_W
  for c in range(_ROWS_PER_W // _CH):
    off = base + c * _CH
    pltpu.sync_copy(idx_hbm.at[pl.ds(off, _CH)], idx_v)
    pltpu.async_copy(table_hbm.at[idx_v], rows_v, sem).wait()
    pltpu.sync_copy(rows_v, out_hbm.at[pl.ds(off, _CH)])


def _sc_gather(table, idx):
  mesh = plsc.VectorSubcoreMesh(core_axis_name="c", subcore_axis_name="s")
  fn = pl.kernel(
      _sc_gather_body,
      out_type=jax.ShapeDtypeStruct((T, D), jnp.float32),
      mesh=mesh,
      scratch_types=[
          pltpu.VMEM((_CH,), jnp.int32),
          pltpu.VMEM((_CH, D), jnp.float32),
          pltpu.SemaphoreType.DMA,
      ],
  )
  return fn(table, idx)


_BM_MM = 512


def _mm_body(x_ref, w_ref, o_ref):
  x = x_ref[...].astype(jnp.float32)
  o_ref[...] = jnp.dot(x, w_ref[...],
                       preferred_element_type=jnp.float32).astype(jnp.bfloat16)


def _matmul(x, w):
  return pl.pallas_call(
      _mm_body,
      grid=(T // _BM_MM,),
      in_specs=[
          pl.BlockSpec((_BM_MM, D), lambda i: (i, 0)),
          pl.BlockSpec((D, D), lambda i: (0, 0)),
      ],
      out_specs=pl.BlockSpec((_BM_MM, D), lambda i: (i, 0)),
      out_shape=jax.ShapeDtypeStruct((T, D), jnp.bfloat16),
  )(x, w)


_BM_ADJ = 256


def _adj_body(a_ref, s_ref, o_ref):
  a = a_ref[0]
  deg = jnp.clip(jnp.sum(a, axis=1, keepdims=True), 1e-9, None)
  s = s_ref[0].astype(jnp.float32)
  o_ref[0] = jnp.maximum(
      jnp.dot(a / deg, s, preferred_element_type=jnp.float32),
      0.0).astype(jnp.bfloat16)


def _adj_mm(adj, sup):
  return pl.pallas_call(
      _adj_body,
      grid=(B, S // _BM_ADJ),
      in_specs=[
          pl.BlockSpec((1, _BM_ADJ, S), lambda b, i: (b, i, 0)),
          pl.BlockSpec((1, S, D), lambda b, i: (b, 0, 0)),
      ],
      out_specs=pl.BlockSpec((1, _BM_ADJ, D), lambda b, i: (b, i, 0)),
      out_shape=jax.ShapeDtypeStruct((B, S, D), jnp.bfloat16),
  )(adj, sup)


_BM_FUSE = 256


def _gelu(x):
  return x * 0.5 * (1.0 + lax.erf(x * (2.0 ** -0.5)))


def _fuse_body(seq_ref, hs_ref, h2_ref, rw_ref, rb_ref, lng_ref, lnb_ref,
               synw_ref, synb_ref, lsw_ref, lsb_ref, llw_ref, llb_ref,
               semw_ref, semb_ref, cw_ref, cb_ref, o_ref):
  i = pl.program_id(0)
  b = i // (S // _BM_FUSE)
  short = seq_ref[b] <= THRESHOLD

  hs = hs_ref[...]
  h2 = h2_ref[...].astype(jnp.float32)

  x = h2 + hs
  m = jnp.mean(x, axis=1, keepdims=True)
  v = jnp.mean((x - m) ** 2, axis=1, keepdims=True)
  shared = (x - m) * lax.rsqrt(v + 1e-5) * lng_ref[...] + lnb_ref[...]

  rl = jnp.dot(hs, rw_ref[...], preferred_element_type=jnp.float32) \
      + rb_ref[...]
  col = lax.broadcasted_iota(jnp.int32, rl.shape, 1)
  neg = jnp.float32(-1e9)
  rl = jnp.where(jnp.logical_and(col == 4, short), neg, rl)
  rl = jnp.where(jnp.logical_and(col == 3, jnp.logical_not(short)), neg, rl)
  rl = rl - jnp.max(rl, axis=1, keepdims=True)
  e = jnp.exp(rl)
  probs = e / jnp.sum(e, axis=1, keepdims=True)

  def group_max(lo, n):
    mx = probs[:, lo:lo + 1]
    idx = jnp.zeros_like(mx, dtype=jnp.int32)
    for j in range(1, n):
      p = probs[:, lo + j:lo + j + 1]
      idx = jnp.where(p > mx, j, idx)
      mx = jnp.maximum(mx, p)
    return mx, idx

  syn_p, syn_i = group_max(0, 3)
  sem_p, sem_i = group_max(5, 3)
  len_p = jnp.where(short, probs[:, 3:4], probs[:, 4:5])
  tot = syn_p + len_p + sem_p
  w_syn = syn_p / tot
  w_len = len_p / tot
  w_sem = sem_p / tot

  fused = jnp.zeros_like(hs)
  for j in range(3):
    eo = _gelu(jnp.dot(shared, synw_ref[j], preferred_element_type=jnp.float32)
               + synb_ref[j:j + 1])
    fused = fused + jnp.where(syn_i == j, w_syn, 0.0) * eo
  lw = jnp.where(short, lsw_ref[...], llw_ref[...])
  lb = jnp.where(short, lsb_ref[...], llb_ref[...])
  lo_ = _gelu(jnp.dot(hs, lw, preferred_element_type=jnp.float32) + lb)
  fused = fused + w_len * lo_
  for j in range(3):
    eo = _gelu(jnp.dot(hs, semw_ref[j], preferred_element_type=jnp.float32)
               + semb_ref[j:j + 1])
    fused = fused + jnp.where(sem_i == j, w_sem, 0.0) * eo

  o_ref[...] = jnp.dot(fused, cw_ref[...],
                       preferred_element_type=jnp.float32) + cb_ref[...]


def _fuse(seq_lengths, hs, h2, router_W, router_b, ln_g, ln_b, syn_W, syn_b,
          lenS_W, lenS_b, lenL_W, lenL_b, sem_W, sem_b, cls_W, cls_b):
  full = lambda shape: pl.BlockSpec(shape, lambda i: tuple(0 for _ in shape))
  return pl.pallas_call(
      _fuse_body,
      grid=(T // _BM_FUSE,),
      in_specs=[
          pl.BlockSpec(memory_space=pltpu.SMEM),
          pl.BlockSpec((_BM_FUSE, D), lambda i: (i, 0)),
          pl.BlockSpec((_BM_FUSE, D), lambda i: (i, 0)),
          full((D, 8)), full((1, 8)),
          full((1, D)), full((1, D)),
          full((3, D, D)), full((3, D)),
          full((D, D)), full((1, D)),
          full((D, D)), full((1, D)),
          full((3, D, D)), full((3, D)),
          full((D, 2)), full((1, 2)),
      ],
      out_specs=pl.BlockSpec((_BM_FUSE, 2), lambda i: (i, 0)),
      out_shape=jax.ShapeDtypeStruct((T, 2), jnp.float32),
  )(seq_lengths, hs, h2, router_W, router_b.reshape(1, 8),
    ln_g.reshape(1, D), ln_b.reshape(1, D), syn_W, syn_b,
    lenS_W, lenS_b.reshape(1, D), lenL_W, lenL_b.reshape(1, D),
    sem_W, sem_b, cls_W, cls_b.reshape(1, 2))


def kernel(input_ids, attention_mask, seq_lengths, adj_matrix, emb, router_W,
           router_b, gcn1_W, gcn2_W, ln_g, ln_b, syn_W, syn_b, lenS_W, lenS_b,
           lenL_W, lenL_b, sem_W, sem_b, cls_W, cls_b):
  del attention_mask
  ids = input_ids.reshape(T).astype(jnp.int32)
  hs = _sc_gather(emb, ids)
  sup1 = _matmul(hs, gcn1_W)
  h1 = _adj_mm(adj_matrix, sup1.reshape(B, S, D))
  sup2 = _matmul(h1.reshape(T, D), gcn2_W)
  h2 = _adj_mm(adj_matrix, sup2.reshape(B, S, D))
  logits = _fuse(seq_lengths.astype(jnp.int32), hs, h2.reshape(T, D),
                 router_W, router_b, ln_g, ln_b, syn_W, syn_b,
                 lenS_W, lenS_b, lenL_W, lenL_b, sem_W, sem_b, cls_W, cls_b)
  return logits.reshape(B, S, 2)


# speedup vs baseline: 1.1884x; 1.0365x over previous
import functools

import jax
import jax.numpy as jnp
from jax import lax
from jax.experimental import pallas as pl
from jax.experimental.pallas import tpu as pltpu
from jax.experimental.pallas import tpu_sc as plsc

B, S, D, V = 2, 2048, 1024, 30000
T = B * S
THRESHOLD = 128

_NC, _NS = 2, 16
_NW = _NC * _NS
_ROWS_PER_W = T // _NW
_CH = 64


def _sc_gather_body(table_hbm, idx_hbm, out_hbm, idx_v, rows_v, sem):
  wid = lax.axis_index("s") * _NC + lax.axis_index("c")
  base = wid * _ROWS_PER_W
  for c in range(_ROWS_PER_W // _CH):
    off = base + c * _CH
    pltpu.sync_copy(idx_hbm.at[pl.ds(off, _CH)], idx_v)
    pltpu.async_copy(table_hbm.at[idx_v], rows_v, sem).wait()
    pltpu.sync_copy(rows_v, out_hbm.at[pl.ds(off, _CH)])


def _sc_gather(table, idx):
  mesh = plsc.VectorSubcoreMesh(core_axis_name="c", subcore_axis_name="s")
  fn = pl.kernel(
      _sc_gather_body,
      out_type=jax.ShapeDtypeStruct((T, D), jnp.float32),
      mesh=mesh,
      scratch_types=[
          pltpu.VMEM((_CH,), jnp.int32),
          pltpu.VMEM((_CH, D), jnp.float32),
          pltpu.SemaphoreType.DMA,
      ],
  )
  return fn(table, idx)


_BM_MM = 512


def _mm_body(x_ref, w_ref, o_ref):
  x = x_ref[...].astype(jnp.float32)
  o_ref[...] = jnp.dot(x, w_ref[...],
                       preferred_element_type=jnp.float32).astype(jnp.bfloat16)


def _matmul(x, w):
  return pl.pallas_call(
      _mm_body,
      grid=(T // _BM_MM,),
      in_specs=[
          pl.BlockSpec((_BM_MM, D), lambda i: (i, 0)),
          pl.BlockSpec((D, D), lambda i: (0, 0)),
      ],
      out_specs=pl.BlockSpec((_BM_MM, D), lambda i: (i, 0)),
      out_shape=jax.ShapeDtypeStruct((T, D), jnp.bfloat16),
  )(x, w)


_BM_ADJ = 256


def _adj_w_body(a_ref, s_ref, w_ref, o_ref):
  a = a_ref[0]
  deg = jnp.clip(jnp.sum(a, axis=1, keepdims=True), 1e-9, None)
  s = s_ref[0].astype(jnp.float32)
  h = jnp.maximum(
      jnp.dot(a / deg, s, preferred_element_type=jnp.float32), 0.0)
  o_ref[0] = jnp.dot(h, w_ref[...],
                     preferred_element_type=jnp.float32).astype(jnp.bfloat16)


def _adj_mm_w(adj, sup, w2):
  return pl.pallas_call(
      _adj_w_body,
      grid=(B, S // _BM_ADJ),
      in_specs=[
          pl.BlockSpec((1, _BM_ADJ, S), lambda b, i: (b, i, 0)),
          pl.BlockSpec((1, S, D), lambda b, i: (b, 0, 0)),
          pl.BlockSpec((D, D), lambda b, i: (0, 0)),
      ],
      out_specs=pl.BlockSpec((1, _BM_ADJ, D), lambda b, i: (b, i, 0)),
      out_shape=jax.ShapeDtypeStruct((B, S, D), jnp.bfloat16),
  )(adj, sup, w2)


def _adj_body(a_ref, s_ref, o_ref):
  a = a_ref[0]
  deg = jnp.clip(jnp.sum(a, axis=1, keepdims=True), 1e-9, None)
  s = s_ref[0].astype(jnp.float32)
  o_ref[0] = jnp.maximum(
      jnp.dot(a / deg, s, preferred_element_type=jnp.float32),
      0.0).astype(jnp.bfloat16)


def _adj_mm(adj, sup):
  return pl.pallas_call(
      _adj_body,
      grid=(B, S // _BM_ADJ),
      in_specs=[
          pl.BlockSpec((1, _BM_ADJ, S), lambda b, i: (b, i, 0)),
          pl.BlockSpec((1, S, D), lambda b, i: (b, 0, 0)),
      ],
      out_specs=pl.BlockSpec((1, _BM_ADJ, D), lambda b, i: (b, i, 0)),
      out_shape=jax.ShapeDtypeStruct((B, S, D), jnp.bfloat16),
  )(adj, sup)


_BM_FUSE = 256


def _gelu(x):
  return x * 0.5 * (1.0 + lax.erf(x * (2.0 ** -0.5)))


def _fuse_body(seq_ref, hs_ref, h2_ref, rw_ref, rb_ref, lng_ref, lnb_ref,
               synw_ref, synb_ref, lsw_ref, lsb_ref, llw_ref, llb_ref,
               semw_ref, semb_ref, cw_ref, cb_ref, o_ref):
  i = pl.program_id(0)
  b = i // (S // _BM_FUSE)
  short = seq_ref[b] <= THRESHOLD

  hs = hs_ref[...]
  h2 = h2_ref[...].astype(jnp.float32)

  x = h2 + hs
  m = jnp.mean(x, axis=1, keepdims=True)
  v = jnp.mean((x - m) ** 2, axis=1, keepdims=True)
  shared = (x - m) * lax.rsqrt(v + 1e-5) * lng_ref[...] + lnb_ref[...]

  rl = jnp.dot(hs, rw_ref[...], preferred_element_type=jnp.float32) \
      + rb_ref[...]
  col = lax.broadcasted_iota(jnp.int32, rl.shape, 1)
  neg = jnp.float32(-1e9)
  rl = jnp.where(jnp.logical_and(col == 4, short), neg, rl)
  rl = jnp.where(jnp.logical_and(col == 3, jnp.logical_not(short)), neg, rl)
  rl = rl - jnp.max(rl, axis=1, keepdims=True)
  e = jnp.exp(rl)
  probs = e / jnp.sum(e, axis=1, keepdims=True)

  def group_max(lo, n):
    mx = probs[:, lo:lo + 1]
    idx = jnp.zeros_like(mx, dtype=jnp.int32)
    for j in range(1, n):
      p = probs[:, lo + j:lo + j + 1]
      idx = jnp.where(p > mx, j, idx)
      mx = jnp.maximum(mx, p)
    return mx, idx

  syn_p, syn_i = group_max(0, 3)
  sem_p, sem_i = group_max(5, 3)
  len_p = jnp.where(short, probs[:, 3:4], probs[:, 4:5])
  tot = syn_p + len_p + sem_p
  w_syn = syn_p / tot
  w_len = len_p / tot
  w_sem = sem_p / tot

  fused = jnp.zeros_like(hs)
  for j in range(3):
    eo = _gelu(jnp.dot(shared, synw_ref[j], preferred_element_type=jnp.float32)
               + synb_ref[j:j + 1])
    fused = fused + jnp.where(syn_i == j, w_syn, 0.0) * eo
  lw = jnp.where(short, lsw_ref[...], llw_ref[...])
  lb = jnp.where(short, lsb_ref[...], llb_ref[...])
  lo_ = _gelu(jnp.dot(hs, lw, preferred_element_type=jnp.float32) + lb)
  fused = fused + w_len * lo_
  for j in range(3):
    eo = _gelu(jnp.dot(hs, semw_ref[j], preferred_element_type=jnp.float32)
               + semb_ref[j:j + 1])
    fused = fused + jnp.where(sem_i == j, w_sem, 0.0) * eo

  o_ref[...] = jnp.dot(fused, cw_ref[...],
                       preferred_element_type=jnp.float32) + cb_ref[...]


def _fuse(seq_lengths, hs, h2, router_W, router_b, ln_g, ln_b, syn_W, syn_b,
          lenS_W, lenS_b, lenL_W, lenL_b, sem_W, sem_b, cls_W, cls_b):
  full = lambda shape: pl.BlockSpec(shape, lambda i: tuple(0 for _ in shape))
  return pl.pallas_call(
      _fuse_body,
      grid=(T // _BM_FUSE,),
      in_specs=[
          pl.BlockSpec(memory_space=pltpu.SMEM),
          pl.BlockSpec((_BM_FUSE, D), lambda i: (i, 0)),
          pl.BlockSpec((_BM_FUSE, D), lambda i: (i, 0)),
          full((D, 8)), full((1, 8)),
          full((1, D)), full((1, D)),
          full((3, D, D)), full((3, D)),
          full((D, D)), full((1, D)),
          full((D, D)), full((1, D)),
          full((3, D, D)), full((3, D)),
          full((D, 2)), full((1, 2)),
      ],
      out_specs=pl.BlockSpec((_BM_FUSE, 2), lambda i: (i, 0)),
      out_shape=jax.ShapeDtypeStruct((T, 2), jnp.float32),
  )(seq_lengths, hs, h2, router_W, router_b.reshape(1, 8),
    ln_g.reshape(1, D), ln_b.reshape(1, D), syn_W, syn_b,
    lenS_W, lenS_b.reshape(1, D), lenL_W, lenL_b.reshape(1, D),
    sem_W, sem_b, cls_W, cls_b.reshape(1, 2))


def kernel(input_ids, attention_mask, seq_lengths, adj_matrix, emb, router_W,
           router_b, gcn1_W, gcn2_W, ln_g, ln_b, syn_W, syn_b, lenS_W, lenS_b,
           lenL_W, lenL_b, sem_W, sem_b, cls_W, cls_b):
  del attention_mask
  ids = input_ids.reshape(T).astype(jnp.int32)
  hs = _sc_gather(emb, ids)
  sup1 = _matmul(hs, gcn1_W)
  sup2 = _adj_mm_w(adj_matrix, sup1.reshape(B, S, D), gcn2_W)
  h2 = _adj_mm(adj_matrix, sup2)
  logits = _fuse(seq_lengths.astype(jnp.int32), hs, h2.reshape(T, D),
                 router_W, router_b, ln_g, ln_b, syn_W, syn_b,
                 lenS_W, lenS_b, lenL_W, lenL_b, sem_W, sem_b, cls_W, cls_b)
  return logits.reshape(B, S, 2)


# speedup vs baseline: 1.2217x; 1.0280x over previous
import functools

import jax
import jax.numpy as jnp
from jax import lax
from jax.experimental import pallas as pl
from jax.experimental.pallas import tpu as pltpu
from jax.experimental.pallas import tpu_sc as plsc

B, S, D, V = 2, 2048, 1024, 30000
T = B * S
THRESHOLD = 128

_NC, _NS = 2, 16
_NW = _NC * _NS
_ROWS_PER_W = T // _NW
_CH = 64


def _sc_gather_body(table_hbm, idx_hbm, out_hbm, idx_v, rows_v, sem):
  wid = lax.axis_index("s") * _NC + lax.axis_index("c")
  base = wid * _ROWS_PER_W
  for c in range(_ROWS_PER_W // _CH):
    off = base + c * _CH
    pltpu.sync_copy(idx_hbm.at[pl.ds(off, _CH)], idx_v)
    pltpu.async_copy(table_hbm.at[idx_v], rows_v, sem).wait()
    pltpu.sync_copy(rows_v, out_hbm.at[pl.ds(off, _CH)])


def _sc_gather(table, idx):
  mesh = plsc.VectorSubcoreMesh(core_axis_name="c", subcore_axis_name="s")
  fn = pl.kernel(
      _sc_gather_body,
      out_type=jax.ShapeDtypeStruct((T, D), jnp.float32),
      mesh=mesh,
      scratch_types=[
          pltpu.VMEM((_CH,), jnp.int32),
          pltpu.VMEM((_CH, D), jnp.float32),
          pltpu.SemaphoreType.DMA,
      ],
  )
  return fn(table, idx)


_BM_MM = 512


def _mm_body(x_ref, w_ref, o_ref):
  x = x_ref[...].astype(jnp.float32)
  o_ref[...] = jnp.dot(x, w_ref[...],
                       preferred_element_type=jnp.float32).astype(jnp.bfloat16)


def _matmul(x, w):
  return pl.pallas_call(
      _mm_body,
      grid=(T // _BM_MM,),
      in_specs=[
          pl.BlockSpec((_BM_MM, D), lambda i: (i, 0)),
          pl.BlockSpec((D, D), lambda i: (0, 0)),
      ],
      out_specs=pl.BlockSpec((_BM_MM, D), lambda i: (i, 0)),
      out_shape=jax.ShapeDtypeStruct((T, D), jnp.bfloat16),
  )(x, w)


_BM_ADJ = 512


def _adj_w_body(a_ref, s_ref, w_ref, o_ref):
  a = a_ref[0]
  deg = jnp.clip(jnp.sum(a, axis=1, keepdims=True), 1e-9, None)
  s = s_ref[0].astype(jnp.float32)
  h = jnp.maximum(
      jnp.dot(a / deg, s, preferred_element_type=jnp.float32), 0.0)
  o_ref[0] = jnp.dot(h, w_ref[...],
                     preferred_element_type=jnp.float32).astype(jnp.bfloat16)


def _adj_mm_w(adj, sup, w2):
  return pl.pallas_call(
      _adj_w_body,
      grid=(B, S // _BM_ADJ),
      in_specs=[
          pl.BlockSpec((1, _BM_ADJ, S), lambda b, i: (b, i, 0)),
          pl.BlockSpec((1, S, D), lambda b, i: (b, 0, 0)),
          pl.BlockSpec((D, D), lambda b, i: (0, 0)),
      ],
      out_specs=pl.BlockSpec((1, _BM_ADJ, D), lambda b, i: (b, i, 0)),
      out_shape=jax.ShapeDtypeStruct((B, S, D), jnp.bfloat16),
  )(adj, sup, w2)


def _adj_body(a_ref, s_ref, o_ref):
  a = a_ref[0]
  deg = jnp.clip(jnp.sum(a, axis=1, keepdims=True), 1e-9, None)
  s = s_ref[0].astype(jnp.float32)
  o_ref[0] = jnp.maximum(
      jnp.dot(a / deg, s, preferred_element_type=jnp.float32),
      0.0).astype(jnp.bfloat16)


def _adj_mm(adj, sup):
  return pl.pallas_call(
      _adj_body,
      grid=(B, S // _BM_ADJ),
      in_specs=[
          pl.BlockSpec((1, _BM_ADJ, S), lambda b, i: (b, i, 0)),
          pl.BlockSpec((1, S, D), lambda b, i: (b, 0, 0)),
      ],
      out_specs=pl.BlockSpec((1, _BM_ADJ, D), lambda b, i: (b, i, 0)),
      out_shape=jax.ShapeDtypeStruct((B, S, D), jnp.bfloat16),
  )(adj, sup)


_BM_FUSE = 256


def _gelu(x):
  return x * 0.5 * (1.0 + lax.erf(x * (2.0 ** -0.5)))


def _fuse_body(seq_ref, hs_ref, h2_ref, rw_ref, rb_ref, lng_ref, lnb_ref,
               synw_ref, synb_ref, lsw_ref, lsb_ref, llw_ref, llb_ref,
               semw_ref, semb_ref, cw_ref, cb_ref, o_ref):
  i = pl.program_id(0)
  b = i // (S // _BM_FUSE)
  short = seq_ref[b] <= THRESHOLD

  hs = hs_ref[...]
  h2 = h2_ref[...].astype(jnp.float32)

  x = h2 + hs
  m = jnp.mean(x, axis=1, keepdims=True)
  v = jnp.mean((x - m) ** 2, axis=1, keepdims=True)
  shared = (x - m) * lax.rsqrt(v + 1e-5) * lng_ref[...] + lnb_ref[...]

  rl = jnp.dot(hs, rw_ref[...], preferred_element_type=jnp.float32) \
      + rb_ref[...]
  col = lax.broadcasted_iota(jnp.int32, rl.shape, 1)
  neg = jnp.float32(-1e9)
  rl = jnp.where(jnp.logical_and(col == 4, short), neg, rl)
  rl = jnp.where(jnp.logical_and(col == 3, jnp.logical_not(short)), neg, rl)
  rl = rl - jnp.max(rl, axis=1, keepdims=True)
  e = jnp.exp(rl)
  probs = e / jnp.sum(e, axis=1, keepdims=True)

  def group_max(lo, n):
    mx = probs[:, lo:lo + 1]
    idx = jnp.zeros_like(mx, dtype=jnp.int32)
    for j in range(1, n):
      p = probs[:, lo + j:lo + j + 1]
      idx = jnp.where(p > mx, j, idx)
      mx = jnp.maximum(mx, p)
    return mx, idx

  syn_p, syn_i = group_max(0, 3)
  sem_p, sem_i = group_max(5, 3)
  len_p = jnp.where(short, probs[:, 3:4], probs[:, 4:5])
  tot = syn_p + len_p + sem_p
  w_syn = syn_p / tot
  w_len = len_p / tot
  w_sem = sem_p / tot

  fused = jnp.zeros_like(hs)
  for j in range(3):
    eo = _gelu(jnp.dot(shared, synw_ref[j], preferred_element_type=jnp.float32)
               + synb_ref[j:j + 1])
    fused = fused + jnp.where(syn_i == j, w_syn, 0.0) * eo
  lw = jnp.where(short, lsw_ref[...], llw_ref[...])
  lb = jnp.where(short, lsb_ref[...], llb_ref[...])
  lo_ = _gelu(jnp.dot(hs, lw, preferred_element_type=jnp.float32) + lb)
  fused = fused + w_len * lo_
  for j in range(3):
    eo = _gelu(jnp.dot(hs, semw_ref[j], preferred_element_type=jnp.float32)
               + semb_ref[j:j + 1])
    fused = fused + jnp.where(sem_i == j, w_sem, 0.0) * eo

  o_ref[...] = jnp.dot(fused, cw_ref[...],
                       preferred_element_type=jnp.float32) + cb_ref[...]


def _fuse(seq_lengths, hs, h2, router_W, router_b, ln_g, ln_b, syn_W, syn_b,
          lenS_W, lenS_b, lenL_W, lenL_b, sem_W, sem_b, cls_W, cls_b):
  full = lambda shape: pl.BlockSpec(shape, lambda i: tuple(0 for _ in shape))
  return pl.pallas_call(
      _fuse_body,
      grid=(T // _BM_FUSE,),
      in_specs=[
          pl.BlockSpec(memory_space=pltpu.SMEM),
          pl.BlockSpec((_BM_FUSE, D), lambda i: (i, 0)),
          pl.BlockSpec((_BM_FUSE, D), lambda i: (i, 0)),
          full((D, 8)), full((1, 8)),
          full((1, D)), full((1, D)),
          full((3, D, D)), full((3, D)),
          full((D, D)), full((1, D)),
          full((D, D)), full((1, D)),
          full((3, D, D)), full((3, D)),
          full((D, 2)), full((1, 2)),
      ],
      out_specs=pl.BlockSpec((_BM_FUSE, 2), lambda i: (i, 0)),
      out_shape=jax.ShapeDtypeStruct((T, 2), jnp.float32),
  )(seq_lengths, hs, h2, router_W, router_b.reshape(1, 8),
    ln_g.reshape(1, D), ln_b.reshape(1, D), syn_W, syn_b,
    lenS_W, lenS_b.reshape(1, D), lenL_W, lenL_b.reshape(1, D),
    sem_W, sem_b, cls_W, cls_b.reshape(1, 2))


def kernel(input_ids, attention_mask, seq_lengths, adj_matrix, emb, router_W,
           router_b, gcn1_W, gcn2_W, ln_g, ln_b, syn_W, syn_b, lenS_W, lenS_b,
           lenL_W, lenL_b, sem_W, sem_b, cls_W, cls_b):
  del attention_mask
  ids = input_ids.reshape(T).astype(jnp.int32)
  hs = _sc_gather(emb, ids)
  sup1 = _matmul(hs, gcn1_W)
  sup2 = _adj_mm_w(adj_matrix, sup1.reshape(B, S, D), gcn2_W)
  h2 = _adj_mm(adj_matrix, sup2)
  logits = _fuse(seq_lengths.astype(jnp.int32), hs, h2.reshape(T, D),
                 router_W, router_b, ln_g, ln_b, syn_W, syn_b,
                 lenS_W, lenS_b, lenL_W, lenL_b, sem_W, sem_b, cls_W, cls_b)
  return logits.reshape(B, S, 2)


# speedup vs baseline: 1.2218x; 1.0001x over previous
import functools

import jax
import jax.numpy as jnp
from jax import lax
from jax.experimental import pallas as pl
from jax.experimental.pallas import tpu as pltpu
from jax.experimental.pallas import tpu_sc as plsc

B, S, D, V = 2, 2048, 1024, 30000
T = B * S
THRESHOLD = 128

_NC, _NS = 2, 16
_NW = _NC * _NS
_ROWS_PER_W = T // _NW
_CH = 64


def _sc_gather_body(table_hbm, idx_hbm, out_hbm, idx_v, rows_v, sem):
  wid = lax.axis_index("s") * _NC + lax.axis_index("c")
  base = wid * _ROWS_PER_W
  for c in range(_ROWS_PER_W // _CH):
    off = base + c * _CH
    pltpu.sync_copy(idx_hbm.at[pl.ds(off, _CH)], idx_v)
    pltpu.async_copy(table_hbm.at[idx_v], rows_v, sem).wait()
    pltpu.sync_copy(rows_v, out_hbm.at[pl.ds(off, _CH)])


def _sc_gather(table, idx):
  mesh = plsc.VectorSubcoreMesh(core_axis_name="c", subcore_axis_name="s")
  fn = pl.kernel(
      _sc_gather_body,
      out_type=jax.ShapeDtypeStruct((T, D), jnp.float32),
      mesh=mesh,
      scratch_types=[
          pltpu.VMEM((_CH,), jnp.int32),
          pltpu.VMEM((_CH, D), jnp.float32),
          pltpu.SemaphoreType.DMA,
      ],
  )
  return fn(table, idx)


_BM_MM = 512


def _mm_body(x_ref, w_ref, o_ref):
  x = x_ref[...].astype(jnp.float32)
  o_ref[...] = jnp.dot(x, w_ref[...],
                       preferred_element_type=jnp.float32).astype(jnp.bfloat16)


def _matmul(x, w):
  return pl.pallas_call(
      _mm_body,
      grid=(T // _BM_MM,),
      in_specs=[
          pl.BlockSpec((_BM_MM, D), lambda i: (i, 0)),
          pl.BlockSpec((D, D), lambda i: (0, 0)),
      ],
      out_specs=pl.BlockSpec((_BM_MM, D), lambda i: (i, 0)),
      out_shape=jax.ShapeDtypeStruct((T, D), jnp.bfloat16),
  )(x, w)


_BM_ADJ = 1024


def _adj_w_body(a_ref, s_ref, w_ref, o_ref):
  a = a_ref[0]
  deg = jnp.clip(jnp.sum(a, axis=1, keepdims=True), 1e-9, None)
  s = s_ref[0].astype(jnp.float32)
  h = jnp.maximum(
      jnp.dot(a / deg, s, preferred_element_type=jnp.float32), 0.0)
  o_ref[0] = jnp.dot(h, w_ref[...],
                     preferred_element_type=jnp.float32).astype(jnp.bfloat16)


def _adj_mm_w(adj, sup, w2):
  return pl.pallas_call(
      _adj_w_body,
      grid=(B, S // _BM_ADJ),
      in_specs=[
          pl.BlockSpec((1, _BM_ADJ, S), lambda b, i: (b, i, 0)),
          pl.BlockSpec((1, S, D), lambda b, i: (b, 0, 0)),
          pl.BlockSpec((D, D), lambda b, i: (0, 0)),
      ],
      out_specs=pl.BlockSpec((1, _BM_ADJ, D), lambda b, i: (b, i, 0)),
      out_shape=jax.ShapeDtypeStruct((B, S, D), jnp.bfloat16),
  )(adj, sup, w2)


def _adj_body(a_ref, s_ref, o_ref):
  a = a_ref[0]
  deg = jnp.clip(jnp.sum(a, axis=1, keepdims=True), 1e-9, None)
  s = s_ref[0].astype(jnp.float32)
  o_ref[0] = jnp.maximum(
      jnp.dot(a / deg, s, preferred_element_type=jnp.float32),
      0.0).astype(jnp.bfloat16)


def _adj_mm(adj, sup):
  return pl.pallas_call(
      _adj_body,
      grid=(B, S // _BM_ADJ),
      in_specs=[
          pl.BlockSpec((1, _BM_ADJ, S), lambda b, i: (b, i, 0)),
          pl.BlockSpec((1, S, D), lambda b, i: (b, 0, 0)),
      ],
      out_specs=pl.BlockSpec((1, _BM_ADJ, D), lambda b, i: (b, i, 0)),
      out_shape=jax.ShapeDtypeStruct((B, S, D), jnp.bfloat16),
  )(adj, sup)


_BM_FUSE = 256


def _gelu(x):
  return x * 0.5 * (1.0 + lax.erf(x * (2.0 ** -0.5)))


def _fuse_body(seq_ref, hs_ref, h2_ref, rw_ref, rb_ref, lng_ref, lnb_ref,
               synw_ref, synb_ref, lsw_ref, lsb_ref, llw_ref, llb_ref,
               semw_ref, semb_ref, cw_ref, cb_ref, o_ref):
  i = pl.program_id(0)
  b = i // (S // _BM_FUSE)
  short = seq_ref[b] <= THRESHOLD

  hs = hs_ref[...]
  h2 = h2_ref[...].astype(jnp.float32)

  x = h2 + hs
  m = jnp.mean(x, axis=1, keepdims=True)
  v = jnp.mean((x - m) ** 2, axis=1, keepdims=True)
  shared = (x - m) * lax.rsqrt(v + 1e-5) * lng_ref[...] + lnb_ref[...]

  rl = jnp.dot(hs, rw_ref[...], preferred_element_type=jnp.float32) \
      + rb_ref[...]
  col = lax.broadcasted_iota(jnp.int32, rl.shape, 1)
  neg = jnp.float32(-1e9)
  rl = jnp.where(jnp.logical_and(col == 4, short), neg, rl)
  rl = jnp.where(jnp.logical_and(col == 3, jnp.logical_not(short)), neg, rl)
  rl = rl - jnp.max(rl, axis=1, keepdims=True)
  e = jnp.exp(rl)
  probs = e / jnp.sum(e, axis=1, keepdims=True)

  def group_max(lo, n):
    mx = probs[:, lo:lo + 1]
    idx = jnp.zeros_like(mx, dtype=jnp.int32)
    for j in range(1, n):
      p = probs[:, lo + j:lo + j + 1]
      idx = jnp.where(p > mx, j, idx)
      mx = jnp.maximum(mx, p)
    return mx, idx

  syn_p, syn_i = group_max(0, 3)
  sem_p, sem_i = group_max(5, 3)
  len_p = jnp.where(short, probs[:, 3:4], probs[:, 4:5])
  tot = syn_p + len_p + sem_p
  w_syn = syn_p / tot
  w_len = len_p / tot
  w_sem = sem_p / tot

  fused = jnp.zeros_like(hs)
  for j in range(3):
    eo = _gelu(jnp.dot(shared, synw_ref[j], preferred_element_type=jnp.float32)
               + synb_ref[j:j + 1])
    fused = fused + jnp.where(syn_i == j, w_syn, 0.0) * eo
  lw = jnp.where(short, lsw_ref[...], llw_ref[...])
  lb = jnp.where(short, lsb_ref[...], llb_ref[...])
  lo_ = _gelu(jnp.dot(hs, lw, preferred_element_type=jnp.float32) + lb)
  fused = fused + w_len * lo_
  for j in range(3):
    eo = _gelu(jnp.dot(hs, semw_ref[j], preferred_element_type=jnp.float32)
               + semb_ref[j:j + 1])
    fused = fused + jnp.where(sem_i == j, w_sem, 0.0) * eo

  o_ref[...] = jnp.dot(fused, cw_ref[...],
                       preferred_element_type=jnp.float32) + cb_ref[...]


def _fuse(seq_lengths, hs, h2, router_W, router_b, ln_g, ln_b, syn_W, syn_b,
          lenS_W, lenS_b, lenL_W, lenL_b, sem_W, sem_b, cls_W, cls_b):
  full = lambda shape: pl.BlockSpec(shape, lambda i: tuple(0 for _ in shape))
  return pl.pallas_call(
      _fuse_body,
      grid=(T // _BM_FUSE,),
      in_specs=[
          pl.BlockSpec(memory_space=pltpu.SMEM),
          pl.BlockSpec((_BM_FUSE, D), lambda i: (i, 0)),
          pl.BlockSpec((_BM_FUSE, D), lambda i: (i, 0)),
          full((D, 8)), full((1, 8)),
          full((1, D)), full((1, D)),
          full((3, D, D)), full((3, D)),
          full((D, D)), full((1, D)),
          full((D, D)), full((1, D)),
          full((3, D, D)), full((3, D)),
          full((D, 2)), full((1, 2)),
      ],
      out_specs=pl.BlockSpec((_BM_FUSE, 2), lambda i: (i, 0)),
      out_shape=jax.ShapeDtypeStruct((T, 2), jnp.float32),
  )(seq_lengths, hs, h2, router_W, router_b.reshape(1, 8),
    ln_g.reshape(1, D), ln_b.reshape(1, D), syn_W, syn_b,
    lenS_W, lenS_b.reshape(1, D), lenL_W, lenL_b.reshape(1, D),
    sem_W, sem_b, cls_W, cls_b.reshape(1, 2))


def kernel(input_ids, attention_mask, seq_lengths, adj_matrix, emb, router_W,
           router_b, gcn1_W, gcn2_W, ln_g, ln_b, syn_W, syn_b, lenS_W, lenS_b,
           lenL_W, lenL_b, sem_W, sem_b, cls_W, cls_b):
  del attention_mask
  ids = input_ids.reshape(T).astype(jnp.int32)
  hs = _sc_gather(emb, ids)
  sup1 = _matmul(hs, gcn1_W)
  sup2 = _adj_mm_w(adj_matrix, sup1.reshape(B, S, D), gcn2_W)
  h2 = _adj_mm(adj_matrix, sup2)
  logits = _fuse(seq_lengths.astype(jnp.int32), hs, h2.reshape(T, D),
                 router_W, router_b, ln_g, ln_b, syn_W, syn_b,
                 lenS_W, lenS_b, lenL_W, lenL_b, sem_W, sem_b, cls_W, cls_b)
  return logits.reshape(B, S, 2)


# speedup vs baseline: 1.2889x; 1.0549x over previous
import functools

import jax
import jax.numpy as jnp
from jax import lax
from jax.experimental import pallas as pl
from jax.experimental.pallas import tpu as pltpu
from jax.experimental.pallas import tpu_sc as plsc

B, S, D, V = 2, 2048, 1024, 30000
T = B * S
THRESHOLD = 128

_NC, _NS = 2, 16
_NW = _NC * _NS
_ROWS_PER_W = T // _NW
_CH = 64


def _sc_gather_body(table_hbm, idx_hbm, out_hbm, idx_v, rows_v, sem):
  wid = lax.axis_index("s") * _NC + lax.axis_index("c")
  base = wid * _ROWS_PER_W
  for c in range(_ROWS_PER_W // _CH):
    off = base + c * _CH
    pltpu.sync_copy(idx_hbm.at[pl.ds(off, _CH)], idx_v)
    pltpu.async_copy(table_hbm.at[idx_v], rows_v, sem).wait()
    pltpu.sync_copy(rows_v, out_hbm.at[pl.ds(off, _CH)])


def _sc_gather(table, idx):
  mesh = plsc.VectorSubcoreMesh(core_axis_name="c", subcore_axis_name="s")
  fn = pl.kernel(
      _sc_gather_body,
      out_type=jax.ShapeDtypeStruct((T, D), jnp.float32),
      mesh=mesh,
      scratch_types=[
          pltpu.VMEM((_CH,), jnp.int32),
          pltpu.VMEM((_CH, D), jnp.float32),
          pltpu.SemaphoreType.DMA,
      ],
  )
  return fn(table, idx)


_BM_MM = 512


def _mm_body(x_ref, w_ref, o_ref):
  x = x_ref[...].astype(jnp.float32)
  o_ref[...] = jnp.dot(x, w_ref[...],
                       preferred_element_type=jnp.float32).astype(jnp.bfloat16)


def _matmul(x, w):
  return pl.pallas_call(
      _mm_body,
      grid=(T // _BM_MM,),
      in_specs=[
          pl.BlockSpec((_BM_MM, D), lambda i: (i, 0)),
          pl.BlockSpec((D, D), lambda i: (0, 0)),
      ],
      out_specs=pl.BlockSpec((_BM_MM, D), lambda i: (i, 0)),
      out_shape=jax.ShapeDtypeStruct((T, D), jnp.bfloat16),
  )(x, w)


_BM_ADJ = 512


def _adj_w_body(a_ref, s_ref, w_ref, o_ref):
  a = a_ref[0]
  deg = jnp.clip(jnp.sum(a, axis=1, keepdims=True), 1e-9, None)
  s = s_ref[0].astype(jnp.float32)
  h = jnp.maximum(
      jnp.dot(a / deg, s, preferred_element_type=jnp.float32), 0.0)
  o_ref[0] = jnp.dot(h, w_ref[...],
                     preferred_element_type=jnp.float32).astype(jnp.bfloat16)


def _adj_mm_w(adj, sup, w2):
  return pl.pallas_call(
      _adj_w_body,
      grid=(B, S // _BM_ADJ),
      in_specs=[
          pl.BlockSpec((1, _BM_ADJ, S), lambda b, i: (b, i, 0)),
          pl.BlockSpec((1, S, D), lambda b, i: (b, 0, 0)),
          pl.BlockSpec((D, D), lambda b, i: (0, 0)),
      ],
      out_specs=pl.BlockSpec((1, _BM_ADJ, D), lambda b, i: (b, i, 0)),
      out_shape=jax.ShapeDtypeStruct((B, S, D), jnp.bfloat16),
  )(adj, sup, w2)


def _adj_body(a_ref, s_ref, o_ref):
  a = a_ref[0]
  deg = jnp.clip(jnp.sum(a, axis=1, keepdims=True), 1e-9, None)
  s = s_ref[0].astype(jnp.float32)
  o_ref[0] = jnp.maximum(
      jnp.dot(a / deg, s, preferred_element_type=jnp.float32),
      0.0).astype(jnp.bfloat16)


def _adj_mm(adj, sup):
  return pl.pallas_call(
      _adj_body,
      grid=(B, S // _BM_ADJ),
      in_specs=[
          pl.BlockSpec((1, _BM_ADJ, S), lambda b, i: (b, i, 0)),
          pl.BlockSpec((1, S, D), lambda b, i: (b, 0, 0)),
      ],
      out_specs=pl.BlockSpec((1, _BM_ADJ, D), lambda b, i: (b, i, 0)),
      out_shape=jax.ShapeDtypeStruct((B, S, D), jnp.bfloat16),
  )(adj, sup)


_BM_FUSE = 512


def _gelu(x):
  return x * 0.5 * (1.0 + lax.erf(x * (2.0 ** -0.5)))


def _fuse_body(seq_ref, hs_ref, h2_ref, rw_ref, rb_ref, lng_ref, lnb_ref,
               synw_ref, synb_ref, lsw_ref, lsb_ref, llw_ref, llb_ref,
               semw_ref, semb_ref, cw_ref, cb_ref, o_ref):
  i = pl.program_id(0)
  b = i // (S // _BM_FUSE)
  short = seq_ref[b] <= THRESHOLD

  hs = hs_ref[...]
  h2 = h2_ref[...].astype(jnp.float32)

  x = h2 + hs
  m = jnp.mean(x, axis=1, keepdims=True)
  v = jnp.mean((x - m) ** 2, axis=1, keepdims=True)
  shared = (x - m) * lax.rsqrt(v + 1e-5) * lng_ref[...] + lnb_ref[...]

  rl = jnp.dot(hs, rw_ref[...], preferred_element_type=jnp.float32) \
      + rb_ref[...]
  col = lax.broadcasted_iota(jnp.int32, rl.shape, 1)
  neg = jnp.float32(-1e9)
  rl = jnp.where(jnp.logical_and(col == 4, short), neg, rl)
  rl = jnp.where(jnp.logical_and(col == 3, jnp.logical_not(short)), neg, rl)
  rl = rl - jnp.max(rl, axis=1, keepdims=True)
  e = jnp.exp(rl)
  probs = e / jnp.sum(e, axis=1, keepdims=True)

  def group_max(lo, n):
    mx = probs[:, lo:lo + 1]
    idx = jnp.zeros_like(mx, dtype=jnp.int32)
    for j in range(1, n):
      p = probs[:, lo + j:lo + j + 1]
      idx = jnp.where(p > mx, j, idx)
      mx = jnp.maximum(mx, p)
    return mx, idx

  syn_p, syn_i = group_max(0, 3)
  sem_p, sem_i = group_max(5, 3)
  len_p = jnp.where(short, probs[:, 3:4], probs[:, 4:5])
  tot = syn_p + len_p + sem_p
  w_syn = syn_p / tot
  w_len = len_p / tot
  w_sem = sem_p / tot

  fused = jnp.zeros_like(hs)
  for j in range(3):
    eo = _gelu(jnp.dot(shared, synw_ref[j], preferred_element_type=jnp.float32)
               + synb_ref[j:j + 1])
    fused = fused + jnp.where(syn_i == j, w_syn, 0.0) * eo
  lw = jnp.where(short, lsw_ref[...], llw_ref[...])
  lb = jnp.where(short, lsb_ref[...], llb_ref[...])
  lo_ = _gelu(jnp.dot(hs, lw, preferred_element_type=jnp.float32) + lb)
  fused = fused + w_len * lo_
  for j in range(3):
    eo = _gelu(jnp.dot(hs, semw_ref[j], preferred_element_type=jnp.float32)
               + semb_ref[j:j + 1])
    fused = fused + jnp.where(sem_i == j, w_sem, 0.0) * eo

  o_ref[...] = jnp.dot(fused, cw_ref[...],
                       preferred_element_type=jnp.float32) + cb_ref[...]


def _fuse(seq_lengths, hs, h2, router_W, router_b, ln_g, ln_b, syn_W, syn_b,
          lenS_W, lenS_b, lenL_W, lenL_b, sem_W, sem_b, cls_W, cls_b):
  full = lambda shape: pl.BlockSpec(shape, lambda i: tuple(0 for _ in shape))
  return pl.pallas_call(
      _fuse_body,
      grid=(T // _BM_FUSE,),
      in_specs=[
          pl.BlockSpec(memory_space=pltpu.SMEM),
          pl.BlockSpec((_BM_FUSE, D), lambda i: (i, 0)),
          pl.BlockSpec((_BM_FUSE, D), lambda i: (i, 0)),
          full((D, 8)), full((1, 8)),
          full((1, D)), full((1, D)),
          full((3, D, D)), full((3, D)),
          full((D, D)), full((1, D)),
          full((D, D)), full((1, D)),
          full((3, D, D)), full((3, D)),
          full((D, 2)), full((1, 2)),
      ],
      out_specs=pl.BlockSpec((_BM_FUSE, 2), lambda i: (i, 0)),
      out_shape=jax.ShapeDtypeStruct((T, 2), jnp.float32),
  )(seq_lengths, hs, h2, router_W, router_b.reshape(1, 8),
    ln_g.reshape(1, D), ln_b.reshape(1, D), syn_W, syn_b,
    lenS_W, lenS_b.reshape(1, D), lenL_W, lenL_b.reshape(1, D),
    sem_W, sem_b, cls_W, cls_b.reshape(1, 2))


def kernel(input_ids, attention_mask, seq_lengths, adj_matrix, emb, router_W,
           router_b, gcn1_W, gcn2_W, ln_g, ln_b, syn_W, syn_b, lenS_W, lenS_b,
           lenL_W, lenL_b, sem_W, sem_b, cls_W, cls_b):
  del attention_mask
  ids = input_ids.reshape(T).astype(jnp.int32)
  hs = _sc_gather(emb, ids)
  sup1 = _matmul(hs, gcn1_W)
  sup2 = _adj_mm_w(adj_matrix, sup1.reshape(B, S, D), gcn2_W)
  h2 = _adj_mm(adj_matrix, sup2)
  logits = _fuse(seq_lengths.astype(jnp.int32), hs, h2.reshape(T, D),
                 router_W, router_b, ln_g, ln_b, syn_W, syn_b,
                 lenS_W, lenS_b, lenL_W, lenL_b, sem_W, sem_b, cls_W, cls_b)
  return logits.reshape(B, S, 2)
